# baseline (speedup 1.0000x reference)
import functools

import jax
import jax.numpy as jnp
import numpy as np
from jax import lax
from jax.experimental import pallas as pl
from jax.experimental.pallas import tpu as pltpu
from jax.experimental.pallas import tpu_sc as plsc

N = 50000
N_P = 50176
E = 800000
B = 4
K = 25
H = 12
G = 32
NCLS = 10
CH = B * H
NC, NS, L = 2, 16, 16
NW = NC * NS
ET = E // NW
ECH = 4000
NCH = 3136

_DFT = np.cos(2.0 * np.pi * np.outer(np.arange(H), np.arange(H)) / H).astype(
    np.float32)

_MESH = plsc.VectorSubcoreMesh(
    core_axis_name="c", subcore_axis_name="s", num_cores=NC, num_subcores=NS)


def _worker_id():
  return lax.axis_index("s") * NC + lax.axis_index("c")


def _deg_pack_body(ei_ref, degp_ref, packed_ref, degbuf, rbuf, cbuf, pbuf):
  w = _worker_id()
  base = w * ET

  def zero_body(i, _):
    degbuf[pl.ds(i * L, L)] = jnp.zeros((L,), jnp.float32)
    return _

  lax.fori_loop(0, N_P // L, zero_body, None)

  pltpu.sync_copy(ei_ref.at[pl.ds(base, ET)], rbuf.at[pl.ds(0, ET)])
  pltpu.sync_copy(ei_ref.at[pl.ds(E + base, ET)], cbuf.at[pl.ds(0, ET)])

  ones = jnp.ones((L,), jnp.float32)

  def pack16(r, c):
    return (plsc.bitcast(r, jnp.uint32) << jnp.uint32(16)) | plsc.bitcast(
        c, jnp.uint32)

  nfull = ET // L

  def edge_body(i, _):
    sl = pl.ds(i * L, L)
    r = rbuf[sl]
    c = cbuf[sl]
    plsc.addupdate_scatter(degbuf, [r], ones)
    pbuf[sl] = pack16(r, c)
    return _

  lax.fori_loop(0, nfull, edge_body, None)

  tail = ET - nfull * L
  if tail:
    mask = lax.iota(jnp.int32, L) < tail
    sl = pl.ds(nfull * L, L)
    r = rbuf[sl]
    c = cbuf[sl]
    plsc.addupdate_scatter(degbuf, [r], ones, mask=mask)
    pbuf[sl] = pack16(r, c)

  pltpu.sync_copy(degbuf, degp_ref.at[w])
  pltpu.sync_copy(pbuf.at[pl.ds(0, ET)], packed_ref.at[pl.ds(base, ET)])


_deg_pack = pl.kernel(
    _deg_pack_body,
    out_type=[
        jax.ShapeDtypeStruct((NW, N_P), jnp.float32),
        jax.ShapeDtypeStruct((E,), jnp.uint32),
    ],
    mesh=_MESH,
    scratch_types=[
        pltpu.VMEM((N_P,), jnp.float32),
        pltpu.VMEM((ET + L, ), jnp.int32),
        pltpu.VMEM((ET + L, ), jnp.int32),
        pltpu.VMEM((ET + L, ), jnp.uint32),
    ],
    compiler_params=pltpu.CompilerParams(use_tc_tiling_on_sc=False, needs_layout_passes=False),
)


def _prep_body(x_ref, degp_ref, dft_ref, xfp_ref, q0p_ref, deg_ref, dis2_ref):
  xb = x_ref[...]
  dg = jnp.sum(degp_ref[...], axis=0, keepdims=True)
  deg_ref[...] = dg
  pos = dg > 0.0
  safe = jnp.maximum(dg, 1.0)
  dis2_ref[...] = jnp.where(pos, 1.0 / safe, 0.0)
  dis = jnp.where(pos, lax.rsqrt(safe), 0.0)
  xf = lax.dot_general(xb, dft_ref[...], (((2,), (0,)), ((), ())),
                       preferred_element_type=jnp.float32)
  xft = jnp.transpose(xf, (0, 2, 1))
  nblk = xft.shape[-1]
  xfp_ref[...] = xft.reshape(CH, nblk)
  q0p_ref[...] = (xft * dis.reshape(1, 1, nblk)).reshape(CH, nblk)


_PREP_NBLK = 1024


def _prep(x, degp, dft):
  nb = _PREP_NBLK
  grid = N_P // nb
  return pl.pallas_call(
      _prep_body,
      grid=(grid,),
      in_specs=[
          pl.BlockSpec((B, nb, H), lambda j: (0, j, 0)),
          pl.BlockSpec((NW, nb), lambda j: (0, j)),
          pl.BlockSpec((H, H), lambda j: (0, 0)),
      ],
      out_specs=[
          pl.BlockSpec((CH, nb), lambda j: (0, j)),
          pl.BlockSpec((CH, nb), lambda j: (0, j)),
          pl.BlockSpec((1, nb), lambda j: (0, j)),
          pl.BlockSpec((1, nb), lambda j: (0, j)),
      ],
      out_shape=[
          jax.ShapeDtypeStruct((CH, N_P), jnp.float32),
          jax.ShapeDtypeStruct((CH, N_P), jnp.float32),
          jax.ShapeDtypeStruct((1, N_P), jnp.float32),
          jax.ShapeDtypeStruct((1, N_P), jnp.float32),
      ],
  )(x, degp, dft)


def _cheb_body(q0_ref, pk_ref, dis2_ref, qall_ref, bufP, bufO, ebufA, ebufB,
               q2bA, d2bA, q2bB, d2bB, semEA, semEB, semP, semO, semQA,
               semDA, semQB, semDB):
  w = _worker_id()
  NEC = E // ECH
  UN = 16

  def zero_buf(buf):
    @plsc.parallel_loop(0, N_P // L, 1, unroll=4)
    def _zb(i):
      buf[pl.ds(i * L, L)] = jnp.zeros((L,), jnp.float32)

  def process(ebuf, src, dst):
    @plsc.parallel_loop(0, ECH // L, 1, unroll=UN)
    def _inner(i):
      e = ebuf[pl.ds(i * L, L)]
      r = plsc.bitcast(e >> jnp.uint32(16), jnp.int32)
      c = plsc.bitcast(e & jnp.uint32(0xFFFF), jnp.int32)
      v = plsc.load_gather(src, [r])
      plsc.addupdate_scatter(dst, [c], v)

  def sweep(src, dst):
    pltpu.async_copy(pk_ref.at[pl.ds(0, ECH)], ebufA, semEA)

    def s2(ci2, _):
      c0 = 2 * ci2
      pltpu.async_copy(pk_ref.at[pl.ds((c0 + 1) * ECH, ECH)], ebufB, semEB)
      pltpu.make_async_copy(pk_ref.at[pl.ds(c0 * ECH, ECH)], ebufA,
                            semEA).wait()
      process(ebufA, src, dst)

      @pl.when(ci2 < NEC // 2 - 1)
      def _():
        pltpu.async_copy(pk_ref.at[pl.ds((c0 + 2) * ECH, ECH)], ebufA, semEA)

      pltpu.make_async_copy(pk_ref.at[pl.ds((c0 + 1) * ECH, ECH)], ebufB,
                            semEB).wait()
      process(ebufB, src, dst)
      return _

    lax.fori_loop(0, NEC // 2, s2, None)

  def combine(k, ch, dst):
    f = jnp.where(k == 1, jnp.float32(1.0), jnp.float32(2.0))
    NCC = N_P // NCH

    def start(ci, q2b, d2b, semQ, semD):
      off = ci * NCH
      pltpu.async_copy(qall_ref.at[k - 1, ch, pl.ds(off, NCH)], q2b, semQ)
      pltpu.async_copy(dis2_ref.at[pl.ds(off, NCH)], d2b, semD)

    def wait(ci, q2b, d2b, semQ, semD):
      off = ci * NCH
      pltpu.make_async_copy(qall_ref.at[k - 1, ch, pl.ds(off, NCH)], q2b,
                            semQ).wait()
      pltpu.make_async_copy(dis2_ref.at[pl.ds(off, NCH)], d2b, semD).wait()

    def work(ci, q2b, d2b):
      off = ci * NCH

      @plsc.parallel_loop(0, NCH // L, 1, unroll=4)
      def _ci16(i):
        asl = pl.ds(off + i * L, L)
        bsl = pl.ds(i * L, L)
        a = dst[asl]
        dst[asl] = (-f) * d2b[bsl] * a - q2b[bsl]

    start(0, q2bA, d2bA, semQA, semDA)

    def cc(ci2, _):
      c0 = 2 * ci2
      start(c0 + 1, q2bB, d2bB, semQB, semDB)
      wait(c0, q2bA, d2bA, semQA, semDA)
      work(c0, q2bA, d2bA)

      @pl.when(ci2 < NCC // 2 - 1)
      def _():
        start(c0 + 2, q2bA, d2bA, semQA, semDA)

      wait(c0 + 1, q2bB, d2bB, semQB, semDB)
      work(c0 + 1, q2bB, d2bB)
      return _

    lax.fori_loop(0, NCC // 2, cc, None)

  def channel_work(ch):
    zero_buf(bufO)
    pltpu.sync_copy(bufO, qall_ref.at[0, ch])
    pltpu.sync_copy(q0_ref.at[ch], bufP)
    pltpu.async_copy(bufP, qall_ref.at[1, ch], semP)

    def k2step(j, _):
      for parity in range(2):
        k = 2 * j + 1 + parity
        src, dst = (bufP, bufO) if parity == 0 else (bufO, bufP)
        ssem, dsem = (semP, semO) if parity == 0 else (semO, semP)
        sweep(src, dst)
        combine(k, ch, dst)
        pltpu.async_copy(dst, qall_ref.at[k + 1, ch], dsem)
        pltpu.make_async_copy(src, qall_ref.at[k, ch], ssem).wait()
        zero_buf(src)
      return _

    lax.fori_loop(0, (K - 1) // 2, k2step, None)
    pltpu.make_async_copy(bufP, qall_ref.at[K, ch], semP).wait()

  channel_work(w)

  @pl.when(w < CH - NW)
  def _():
    channel_work(w + NW)


_cheb = pl.kernel(
    _cheb_body,
    out_type=jax.ShapeDtypeStruct((K + 1, CH, N_P), jnp.float32),
    mesh=_MESH,
    scratch_types=[
        pltpu.VMEM((N_P,), jnp.float32),
        pltpu.VMEM((N_P,), jnp.float32),
        pltpu.VMEM((ECH,), jnp.uint32),
        pltpu.VMEM((ECH,), jnp.uint32),
        pltpu.VMEM((NCH,), jnp.float32),
        pltpu.VMEM((NCH,), jnp.float32),
        pltpu.VMEM((NCH,), jnp.float32),
        pltpu.VMEM((NCH,), jnp.float32),
        pltpu.SemaphoreType.DMA,
        pltpu.SemaphoreType.DMA,
        pltpu.SemaphoreType.DMA,
        pltpu.SemaphoreType.DMA,
        pltpu.SemaphoreType.DMA,
        pltpu.SemaphoreType.DMA,
        pltpu.SemaphoreType.DMA,
        pltpu.SemaphoreType.DMA,
    ],
    compiler_params=pltpu.CompilerParams(use_tc_tiling_on_sc=False, needs_layout_passes=False),
)


def _post_body(qall_ref, xfp_ref, deg_ref, degc_ref, w_ref, bias_ref, fcw_ref,
               fcb_ref, out_ref, acc_ref):
  j = pl.program_id(0)
  nblk = deg_ref.shape[-1]
  wfull = w_ref[...]
  sq_row = jnp.sqrt(deg_ref[...])
  iso = degc_ref[...] == 0.0
  qa = qall_ref[...]
  xfb = xfp_ref[...]
  wc = wfull[0] + sum(
      ((-1.0) ** (k // 2)) * wfull[k] for k in range(2, K, 2))
  wr = wfull.reshape(K * H, G)
  nmask = (lax.broadcasted_iota(jnp.int32, (nblk, 1), 0) + j * nblk) < N
  fcw3 = fcw_ref[...]

  @pl.when(j == 0)
  def _():
    acc_ref[...] = jnp.zeros_like(acc_ref)

  for b in range(B):
    tb = qa[1:, b * H:(b + 1) * H, :].reshape(K * H, nblk) * sq_row
    ob = lax.dot_general(tb, wr, (((0,), (0,)), ((), ())),
                         preferred_element_type=jnp.float32)
    cb = lax.dot_general(xfb[b * H:(b + 1) * H, :], wc,
                         (((0,), (0,)), ((), ())),
                         preferred_element_type=jnp.float32)
    ob = jnp.where(iso, cb, ob)
    hb = jnp.maximum(ob + bias_ref[...].reshape(1, G), 0.0)
    hb = jnp.where(nmask, hb, 0.0)
    lp = jnp.concatenate(
        [jnp.sum(hb * fcw3[c], axis=0, keepdims=True) for c in range(NCLS)],
        axis=0)
    acc_ref[b] += lp

  @pl.when(j == pl.num_programs(0) - 1)
  def _():
    lg = jnp.sum(acc_ref[...], axis=2) + fcb_ref[...]
    m = jnp.max(lg, axis=1, keepdims=True)
    ls = lg - m
    lse = jnp.log(jnp.sum(jnp.exp(ls), axis=1, keepdims=True))
    out_ref[...] = ls - lse


_POST_NBLK = 1024


def _post(qall, xfp, deg, w, bias2, fcw, fcb2):
  nb = _POST_NBLK
  grid = N_P // nb
  return pl.pallas_call(
      _post_body,
      grid=(grid,),
      in_specs=[
          pl.BlockSpec((K + 1, CH, nb), lambda j: (0, 0, j)),
          pl.BlockSpec((CH, nb), lambda j: (0, j)),
          pl.BlockSpec((1, nb), lambda j: (0, j)),
          pl.BlockSpec((nb, 1), lambda j: (j, 0)),
          pl.BlockSpec((K, H, G), lambda j: (0, 0, 0)),
          pl.BlockSpec((1, G), lambda j: (0, 0)),
          pl.BlockSpec((NCLS, nb, G), lambda j: (0, j, 0)),
          pl.BlockSpec((1, NCLS), lambda j: (0, 0)),
      ],
      out_specs=pl.BlockSpec((B, NCLS), lambda j: (0, 0)),
      out_shape=jax.ShapeDtypeStruct((B, NCLS), jnp.float32),
      scratch_shapes=[pltpu.VMEM((B, NCLS, G), jnp.float32)],
  )(qall, xfp, deg, deg.reshape(N_P, 1), w, bias2,
    fcw.reshape(NCLS, N, G), fcb2)


def kernel(x, edge_index, W, bias, fc_w, fc_b):
  degp, packed = _deg_pack(edge_index.reshape(2 * E))
  dft = jnp.asarray(_DFT)
  xfp, q0p, deg, dis2 = _prep(x, degp, dft)
  qall = _cheb(q0p, packed, dis2.reshape(N_P))
  return _post(qall, xfp, deg, W, bias.reshape(1, G), fc_w,
               fc_b.reshape(1, NCLS))


# speedup vs baseline: 1.9349x; 1.9349x over previous
import functools

import jax
import jax.numpy as jnp
import numpy as np
from jax import lax
from jax.experimental import pallas as pl
from jax.experimental.pallas import tpu as pltpu
from jax.experimental.pallas import tpu_sc as plsc

N = 50000
N_P = 50176
E = 800000
B = 4
K = 25
H = 12
G = 32
NCLS = 10
HU = 7
CH = B * HU
NC, NS, L = 2, 16, 16
NW = NC * NS
ET = E // NW
ECH = 4000
NCH = 3136

_DFT = np.cos(2.0 * np.pi * np.outer(np.arange(H), np.arange(HU)) / H).astype(
    np.float32)

_MESH = plsc.VectorSubcoreMesh(
    core_axis_name="c", subcore_axis_name="s", num_cores=NC, num_subcores=NS)


def _worker_id():
  return lax.axis_index("s") * NC + lax.axis_index("c")


def _deg_pack_body(ei_ref, degp_ref, packed_ref, degbuf, rbuf, cbuf, pbuf):
  w = _worker_id()
  base = w * ET

  def zero_body(i, _):
    degbuf[pl.ds(i * L, L)] = jnp.zeros((L,), jnp.float32)
    return _

  lax.fori_loop(0, N_P // L, zero_body, None)

  pltpu.sync_copy(ei_ref.at[pl.ds(base, ET)], rbuf.at[pl.ds(0, ET)])
  pltpu.sync_copy(ei_ref.at[pl.ds(E + base, ET)], cbuf.at[pl.ds(0, ET)])

  ones = jnp.ones((L,), jnp.float32)

  def pack16(r, c):
    return (plsc.bitcast(r, jnp.uint32) << jnp.uint32(16)) | plsc.bitcast(
        c, jnp.uint32)

  nfull = ET // L

  def edge_body(i, _):
    sl = pl.ds(i * L, L)
    r = rbuf[sl]
    c = cbuf[sl]
    plsc.addupdate_scatter(degbuf, [r], ones)
    pbuf[sl] = pack16(r, c)
    return _

  lax.fori_loop(0, nfull, edge_body, None)

  tail = ET - nfull * L
  if tail:
    mask = lax.iota(jnp.int32, L) < tail
    sl = pl.ds(nfull * L, L)
    r = rbuf[sl]
    c = cbuf[sl]
    plsc.addupdate_scatter(degbuf, [r], ones, mask=mask)
    pbuf[sl] = pack16(r, c)

  pltpu.sync_copy(degbuf, degp_ref.at[w])
  pltpu.sync_copy(pbuf.at[pl.ds(0, ET)], packed_ref.at[pl.ds(base, ET)])


_deg_pack = pl.kernel(
    _deg_pack_body,
    out_type=[
        jax.ShapeDtypeStruct((NW, N_P), jnp.float32),
        jax.ShapeDtypeStruct((E,), jnp.uint32),
    ],
    mesh=_MESH,
    scratch_types=[
        pltpu.VMEM((N_P,), jnp.float32),
        pltpu.VMEM((ET + L, ), jnp.int32),
        pltpu.VMEM((ET + L, ), jnp.int32),
        pltpu.VMEM((ET + L, ), jnp.uint32),
    ],
    compiler_params=pltpu.CompilerParams(use_tc_tiling_on_sc=False, needs_layout_passes=False),
)


def _prep_body(x_ref, degp_ref, dft_ref, xfp_ref, q0p_ref, deg_ref, dis2_ref):
  xb = x_ref[...]
  dg = jnp.sum(degp_ref[...], axis=0, keepdims=True)
  deg_ref[...] = dg
  pos = dg > 0.0
  safe = jnp.maximum(dg, 1.0)
  dis2_ref[...] = jnp.where(pos, 1.0 / safe, 0.0)
  dis = jnp.where(pos, lax.rsqrt(safe), 0.0)
  xf = lax.dot_general(xb, dft_ref[...], (((2,), (0,)), ((), ())),
                       preferred_element_type=jnp.float32)
  xft = jnp.transpose(xf, (0, 2, 1))
  nblk = xft.shape[-1]
  xfp_ref[...] = xft.reshape(CH, nblk)
  q0p_ref[...] = (xft * dis.reshape(1, 1, nblk)).reshape(CH, nblk)


_PREP_NBLK = 1024


def _prep(x, degp, dft):
  nb = _PREP_NBLK
  grid = N_P // nb
  return pl.pallas_call(
      _prep_body,
      grid=(grid,),
      in_specs=[
          pl.BlockSpec((B, nb, H), lambda j: (0, j, 0)),
          pl.BlockSpec((NW, nb), lambda j: (0, j)),
          pl.BlockSpec((H, HU), lambda j: (0, 0)),
      ],
      out_specs=[
          pl.BlockSpec((CH, nb), lambda j: (0, j)),
          pl.BlockSpec((CH, nb), lambda j: (0, j)),
          pl.BlockSpec((1, nb), lambda j: (0, j)),
          pl.BlockSpec((1, nb), lambda j: (0, j)),
      ],
      out_shape=[
          jax.ShapeDtypeStruct((CH, N_P), jnp.float32),
          jax.ShapeDtypeStruct((CH, N_P), jnp.float32),
          jax.ShapeDtypeStruct((1, N_P), jnp.float32),
          jax.ShapeDtypeStruct((1, N_P), jnp.float32),
      ],
  )(x, degp, dft)


def _cheb_body(q0_ref, pk_ref, dis2_ref, qall_ref, bufP, bufO, ebufA, ebufB,
               q2bA, d2bA, q2bB, d2bB, semEA, semEB, semP, semO, semQA,
               semDA, semQB, semDB):
  w = _worker_id()
  NEC = E // ECH
  UN = 8

  def zero_buf(buf):
    @plsc.parallel_loop(0, N_P // L, 1, unroll=4)
    def _zb(i):
      buf[pl.ds(i * L, L)] = jnp.zeros((L,), jnp.float32)

  def process(ebuf, src, dst):
    @plsc.parallel_loop(0, ECH // L, 1, unroll=UN)
    def _inner(i):
      e = ebuf[pl.ds(i * L, L)]
      r = plsc.bitcast(e >> jnp.uint32(16), jnp.int32)
      c = plsc.bitcast(e & jnp.uint32(0xFFFF), jnp.int32)
      v = plsc.load_gather(src, [r])
      plsc.addupdate_scatter(dst, [c], v)

  def sweep(src, dst):
    pltpu.async_copy(pk_ref.at[pl.ds(0, ECH)], ebufA, semEA)

    def s2(ci2, _):
      c0 = 2 * ci2
      pltpu.async_copy(pk_ref.at[pl.ds((c0 + 1) * ECH, ECH)], ebufB, semEB)
      pltpu.make_async_copy(pk_ref.at[pl.ds(c0 * ECH, ECH)], ebufA,
                            semEA).wait()
      process(ebufA, src, dst)

      @pl.when(ci2 < NEC // 2 - 1)
      def _():
        pltpu.async_copy(pk_ref.at[pl.ds((c0 + 2) * ECH, ECH)], ebufA, semEA)

      pltpu.make_async_copy(pk_ref.at[pl.ds((c0 + 1) * ECH, ECH)], ebufB,
                            semEB).wait()
      process(ebufB, src, dst)
      return _

    lax.fori_loop(0, NEC // 2, s2, None)

  def combine(k, ch, dst):
    f = jnp.where(k == 1, jnp.float32(1.0), jnp.float32(2.0))
    NCC = N_P // NCH

    def start(ci, q2b, d2b, semQ, semD):
      off = ci * NCH
      pltpu.async_copy(qall_ref.at[k - 1, ch, pl.ds(off, NCH)], q2b, semQ)
      pltpu.async_copy(dis2_ref.at[pl.ds(off, NCH)], d2b, semD)

    def wait(ci, q2b, d2b, semQ, semD):
      off = ci * NCH
      pltpu.make_async_copy(qall_ref.at[k - 1, ch, pl.ds(off, NCH)], q2b,
                            semQ).wait()
      pltpu.make_async_copy(dis2_ref.at[pl.ds(off, NCH)], d2b, semD).wait()

    def work(ci, q2b, d2b):
      off = ci * NCH

      @plsc.parallel_loop(0, NCH // L, 1, unroll=4)
      def _ci16(i):
        asl = pl.ds(off + i * L, L)
        bsl = pl.ds(i * L, L)
        a = dst[asl]
        dst[asl] = (-f) * d2b[bsl] * a - q2b[bsl]

    start(0, q2bA, d2bA, semQA, semDA)

    def cc(ci2, _):
      c0 = 2 * ci2
      start(c0 + 1, q2bB, d2bB, semQB, semDB)
      wait(c0, q2bA, d2bA, semQA, semDA)
      work(c0, q2bA, d2bA)

      @pl.when(ci2 < NCC // 2 - 1)
      def _():
        start(c0 + 2, q2bA, d2bA, semQA, semDA)

      wait(c0 + 1, q2bB, d2bB, semQB, semDB)
      work(c0 + 1, q2bB, d2bB)
      return _

    lax.fori_loop(0, NCC // 2, cc, None)

  def channel_work(ch):
    zero_buf(bufO)
    pltpu.sync_copy(bufO, qall_ref.at[0, ch])
    pltpu.sync_copy(q0_ref.at[ch], bufP)
    pltpu.async_copy(bufP, qall_ref.at[1, ch], semP)

    def k2step(j, _):
      for parity in range(2):
        k = 2 * j + 1 + parity
        src, dst = (bufP, bufO) if parity == 0 else (bufO, bufP)
        ssem, dsem = (semP, semO) if parity == 0 else (semO, semP)
        sweep(src, dst)
        combine(k, ch, dst)
        pltpu.async_copy(dst, qall_ref.at[k + 1, ch], dsem)
        pltpu.make_async_copy(src, qall_ref.at[k, ch], ssem).wait()
        zero_buf(src)
      return _

    lax.fori_loop(0, (K - 1) // 2, k2step, None)
    pltpu.make_async_copy(bufP, qall_ref.at[K, ch], semP).wait()

  @pl.when(w < CH)
  def _():
    channel_work(w)


_cheb = pl.kernel(
    _cheb_body,
    out_type=jax.ShapeDtypeStruct((K + 1, CH, N_P), jnp.float32),
    mesh=_MESH,
    scratch_types=[
        pltpu.VMEM((N_P,), jnp.float32),
        pltpu.VMEM((N_P,), jnp.float32),
        pltpu.VMEM((ECH,), jnp.uint32),
        pltpu.VMEM((ECH,), jnp.uint32),
        pltpu.VMEM((NCH,), jnp.float32),
        pltpu.VMEM((NCH,), jnp.float32),
        pltpu.VMEM((NCH,), jnp.float32),
        pltpu.VMEM((NCH,), jnp.float32),
        pltpu.SemaphoreType.DMA,
        pltpu.SemaphoreType.DMA,
        pltpu.SemaphoreType.DMA,
        pltpu.SemaphoreType.DMA,
        pltpu.SemaphoreType.DMA,
        pltpu.SemaphoreType.DMA,
        pltpu.SemaphoreType.DMA,
        pltpu.SemaphoreType.DMA,
    ],
    compiler_params=pltpu.CompilerParams(use_tc_tiling_on_sc=False, needs_layout_passes=False),
)


def _post_body(qall_ref, xfp_ref, deg_ref, degc_ref, w_ref, bias_ref, fcw_ref,
               fcb_ref, out_ref, acc_ref):
  j = pl.program_id(0)
  nblk = deg_ref.shape[-1]
  wfull = w_ref[...]
  sq_row = jnp.sqrt(deg_ref[...])
  iso = degc_ref[...] == 0.0
  qa = qall_ref[...]
  xfb = xfp_ref[...]
  terms = [wfull[:, 0:1, :]]
  for h in range(1, H - HU + 1):
    terms.append(wfull[:, h:h + 1, :] + wfull[:, H - h:H - h + 1, :])
  terms.append(wfull[:, HU - 1:HU, :])
  wp = jnp.concatenate(terms, axis=1)
  wc = wp[0] + sum(
      ((-1.0) ** (k // 2)) * wp[k] for k in range(2, K, 2))
  wr = wp.reshape(K * HU, G)
  nmask = (lax.broadcasted_iota(jnp.int32, (nblk, 1), 0) + j * nblk) < N
  fcw3 = fcw_ref[...]

  @pl.when(j == 0)
  def _():
    acc_ref[...] = jnp.zeros_like(acc_ref)

  for b in range(B):
    tb = qa[1:, b * HU:(b + 1) * HU, :].reshape(K * HU, nblk) * sq_row
    ob = lax.dot_general(tb, wr, (((0,), (0,)), ((), ())),
                         preferred_element_type=jnp.float32)
    cb = lax.dot_general(xfb[b * HU:(b + 1) * HU, :], wc,
                         (((0,), (0,)), ((), ())),
                         preferred_element_type=jnp.float32)
    ob = jnp.where(iso, cb, ob)
    hb = jnp.maximum(ob + bias_ref[...].reshape(1, G), 0.0)
    hb = jnp.where(nmask, hb, 0.0)
    lp = jnp.concatenate(
        [jnp.sum(hb * fcw3[c], axis=0, keepdims=True) for c in range(NCLS)],
        axis=0)
    acc_ref[b] += lp

  @pl.when(j == pl.num_programs(0) - 1)
  def _():
    lg = jnp.sum(acc_ref[...], axis=2) + fcb_ref[...]
    m = jnp.max(lg, axis=1, keepdims=True)
    ls = lg - m
    lse = jnp.log(jnp.sum(jnp.exp(ls), axis=1, keepdims=True))
    out_ref[...] = ls - lse


_POST_NBLK = 1024


def _post(qall, xfp, deg, w, bias2, fcw, fcb2):
  nb = _POST_NBLK
  grid = N_P // nb
  return pl.pallas_call(
      _post_body,
      grid=(grid,),
      in_specs=[
          pl.BlockSpec((K + 1, CH, nb), lambda j: (0, 0, j)),
          pl.BlockSpec((CH, nb), lambda j: (0, j)),
          pl.BlockSpec((1, nb), lambda j: (0, j)),
          pl.BlockSpec((nb, 1), lambda j: (j, 0)),
          pl.BlockSpec((K, H, G), lambda j: (0, 0, 0)),
          pl.BlockSpec((1, G), lambda j: (0, 0)),
          pl.BlockSpec((NCLS, nb, G), lambda j: (0, j, 0)),
          pl.BlockSpec((1, NCLS), lambda j: (0, 0)),
      ],
      out_specs=pl.BlockSpec((B, NCLS), lambda j: (0, 0)),
      out_shape=jax.ShapeDtypeStruct((B, NCLS), jnp.float32),
      scratch_shapes=[pltpu.VMEM((B, NCLS, G), jnp.float32)],
  )(qall, xfp, deg, deg.reshape(N_P, 1), w, bias2,
    fcw.reshape(NCLS, N, G), fcb2)


def kernel(x, edge_index, W, bias, fc_w, fc_b):
  degp, packed = _deg_pack(edge_index.reshape(2 * E))
  dft = jnp.asarray(_DFT)
  xfp, q0p, deg, dis2 = _prep(x, degp, dft)
  qall = _cheb(q0p, packed, dis2.reshape(N_P))
  return _post(qall, xfp, deg, W, bias.reshape(1, G), fc_w,
               fc_b.reshape(1, NCLS))
